# baseline (speedup 1.0000x reference)
import jax
import jax.numpy as jnp
from jax import lax
from jax.experimental import pallas as pl
from jax.experimental.pallas import tpu as pltpu
from jax.experimental.pallas import tpu_sc as plsc

_N = 10000
_E = 160000
_D = 128
_R = 8
_NB = 4

_NP = 10240
_NBLK = 40
_BN = 256
_NC = 2
_NS = 16
_EPW = 5120
_CK = 80
_ECH = _EPW // _CK
_ECH_A = 122
_ECH_B = 6
_NW = _NC * _NS
_EP = _NW * _EPW
_PSB = _NP // _NS


def _a1_body(x_ref, wc_ref, xp_ref):
    xp_ref[...] = jnp.dot(x_ref[...], wc_ref[...],
                          preferred_element_type=jnp.float32)[None]


def _run_a1(xpad, wc):
    return pl.pallas_call(
        _a1_body,
        grid=(_R,),
        in_specs=[
            pl.BlockSpec((_NP, _D), lambda r: (0, 0)),
            pl.BlockSpec((_D, _D), lambda r: (0, r)),
        ],
        out_specs=pl.BlockSpec((1, _NP, _D), lambda r: (r, 0, 0)),
        out_shape=jax.ShapeDtypeStruct((_R, _NP, _D), jnp.float32),
    )(xpad, wc)


def _a2_body(x_ref, ws_ref, aa_ref, xr_ref, s_ref, a_ref, ma_ref, ms_ref):
    i = pl.program_id(0)
    res = jnp.dot(x_ref[...], ws_ref[...], preferred_element_type=jnp.float32)
    xr = res[:, :_D]
    s = res[:, _D:_D + _R]
    a = jnp.dot(xr, aa_ref[...], preferred_element_type=jnp.float32)
    xr_ref[...] = xr
    s_ref[...] = s
    a_ref[...] = a
    bma = jnp.max(a)
    bms = jnp.max(s)

    @pl.when(i == 0)
    def _():
        ma_ref[0, 0] = bma
        ms_ref[0, 0] = bms

    @pl.when(i > 0)
    def _():
        ma_ref[0, 0] = jnp.maximum(ma_ref[0, 0], bma)
        ms_ref[0, 0] = jnp.maximum(ms_ref[0, 0], bms)


def _run_a2(xpad, ws, aa):
    return pl.pallas_call(
        _a2_body,
        grid=(_NBLK,),
        in_specs=[
            pl.BlockSpec((_BN, _D), lambda i: (i, 0)),
            pl.BlockSpec((_D, 2 * _D), lambda i: (0, 0)),
            pl.BlockSpec((_D, 1), lambda i: (0, 0)),
        ],
        out_specs=[
            pl.BlockSpec((_BN, _D), lambda i: (i, 0)),
            pl.BlockSpec((_BN, _R), lambda i: (i, 0)),
            pl.BlockSpec((_BN, 1), lambda i: (i, 0)),
            pl.BlockSpec(memory_space=pltpu.SMEM),
            pl.BlockSpec(memory_space=pltpu.SMEM),
        ],
        out_shape=[
            jax.ShapeDtypeStruct((_NP, _D), jnp.float32),
            jax.ShapeDtypeStruct((_NP, _R), jnp.float32),
            jax.ShapeDtypeStruct((_NP, 1), jnp.float32),
            jax.ShapeDtypeStruct((1, 1), jnp.float32),
            jax.ShapeDtypeStruct((1, 1), jnp.float32),
        ],
    )(xpad, ws, aa)


def _sc_body(xp_hbm, ai_hbm, s_hbm, cidx_hbm, c_hbm,
             aggr_out, den_out,
             ci0, ci1, a_row0, a_row1, s_row0, s_row1, e_row0, e_row1,
             rows_v0, rows_v1, c_v, den_v, aggr_sp,
             sem_a0, sem_a1, sem_s0, sem_s1, sem_r0, sem_r1,
             sem_w0, sem_w1):
    cid = lax.axis_index("c")
    sid = lax.axis_index("s")
    wid = sid * _NC + cid
    ech = jnp.where(cid == 0, _ECH_A, _ECH_B)
    cbase = jnp.where(cid == 0, 0, _ECH_A * _NS) + sid * ech

    cis = (ci0, ci1)
    a_rows = (a_row0, a_row1)
    s_rows = (s_row0, s_row1)
    e_rows = (e_row0, e_row1)
    rows_vs = (rows_v0, rows_v1)
    sems_a = (sem_a0, sem_a1)
    sems_s = (sem_s0, sem_s1)
    sems_r = (sem_r0, sem_r1)
    sems_w = (sem_w0, sem_w1)

    def zrow(i, c0):
        for f in range(8):
            rows_v0[i, pl.ds(f * 16, 16)] = jnp.zeros((16,), jnp.float32)
        return c0

    lax.fori_loop(0, _CK, zrow, 0)

    def zcopy(i, c0):
        pltpu.sync_copy(rows_v0,
                        aggr_sp.at[pl.ds(sid * _PSB + i * _CK, _CK)])
        return c0

    lax.fori_loop(0, _PSB // _CK, zcopy, 0)

    def zden(i, c0):
        den_v[pl.ds(i * 16, 16)] = jnp.zeros((16,), jnp.float32)
        return c0

    lax.fori_loop(0, _NP // 16, zden, 0)

    pltpu.sync_copy(c_hbm, c_v)
    cvec = c_v[...]

    plsc.subcore_barrier()

    def issue(ch, b):
        pltpu.async_copy(ai_hbm.at[cis[b].at[0]], a_rows[b], sems_a[b])
        pltpu.async_copy(s_hbm.at[cis[b].at[1]], s_rows[b], sems_s[b])
        pltpu.async_copy(xp_hbm.at[cis[b].at[2]], rows_vs[b], sems_r[b])

    pltpu.sync_copy(cidx_hbm.at[cbase], ci0)
    issue(0, 0)

    def halfstep(half, carry):
        for b in (0, 1):
            ch = half * 2 + b
            nb = 1 - b

            @pl.when(ch + 1 < ech)
            def _():
                pltpu.sync_copy(cidx_hbm.at[cbase + ch + 1], cis[nb])

                @pl.when(ch >= 1)
                def _():
                    pltpu.make_async_copy(
                        rows_vs[nb], aggr_sp.at[cis[nb].at[0]],
                        sems_w[nb]).wait()

                issue(ch + 1, nb)

            pltpu.make_async_copy(
                ai_hbm.at[cis[b].at[0]], a_rows[b], sems_a[b]).wait()
            pltpu.make_async_copy(
                s_hbm.at[cis[b].at[1]], s_rows[b], sems_s[b]).wait()
            for k in range(_CK // 16):
                a16 = a_rows[b][pl.ds(k * 16, 16)]
                s16 = s_rows[b][pl.ds(k * 16, 16)]
                d16 = cis[b][0, pl.ds(k * 16, 16)]
                z = a16 + s16
                alpha = jnp.where(z >= 0.0, z, 0.2 * z) - cvec
                e16 = jnp.exp(alpha)
                e_rows[b][pl.ds(k * 16, 16)] = e16
                plsc.addupdate_scatter(den_v, [d16], e16)
            pltpu.make_async_copy(
                xp_hbm.at[cis[b].at[2]], rows_vs[b], sems_r[b]).wait()

            def scale_row(j, c2):
                b16 = plsc.load_gather(
                    e_rows[b], [jnp.full((16,), j, jnp.int32)])
                for f in range(8):
                    rows_vs[b][j, pl.ds(f * 16, 16)] = (
                        rows_vs[b][j, pl.ds(f * 16, 16)] * b16)
                return c2

            lax.fori_loop(0, _CK, scale_row, 0)

            pltpu.async_copy(rows_vs[b], aggr_sp.at[cis[b].at[0]],
                             sems_w[b], add=True)
        return carry

    lax.fori_loop(0, (ech + 1) // 2, halfstep, 0)

    pltpu.make_async_copy(rows_vs[0], aggr_sp.at[cis[0].at[0]],
                          sems_w[0]).wait()
    pltpu.make_async_copy(rows_vs[1], aggr_sp.at[cis[1].at[0]],
                          sems_w[1]).wait()

    plsc.subcore_barrier()

    pltpu.sync_copy(aggr_sp.at[pl.ds(sid * _PSB, _PSB)],
                    aggr_out.at[cid, pl.ds(sid * _PSB, _PSB)])
    pltpu.sync_copy(den_v, den_out.at[wid])


def _run_sc(xp, ai, sflat, cidx3d, cvec):
    mesh = plsc.VectorSubcoreMesh(core_axis_name="c", subcore_axis_name="s")
    f = pl.kernel(
        _sc_body,
        out_type=[
            jax.ShapeDtypeStruct((_NC, _NP, _D), jnp.float32),
            jax.ShapeDtypeStruct((_NW, _NP), jnp.float32),
        ],
        mesh=mesh,
        compiler_params=pltpu.CompilerParams(needs_layout_passes=False),
        scratch_types=[
            pltpu.VMEM((3, _CK), jnp.int32),
            pltpu.VMEM((3, _CK), jnp.int32),
            pltpu.VMEM((_CK,), jnp.float32),
            pltpu.VMEM((_CK,), jnp.float32),
            pltpu.VMEM((_CK,), jnp.float32),
            pltpu.VMEM((_CK,), jnp.float32),
            pltpu.VMEM((_CK,), jnp.float32),
            pltpu.VMEM((_CK,), jnp.float32),
            pltpu.VMEM((_CK, _D), jnp.float32),
            pltpu.VMEM((_CK, _D), jnp.float32),
            pltpu.VMEM((16,), jnp.float32),
            pltpu.VMEM((_NP,), jnp.float32),
            pltpu.VMEM_SHARED((_NP, _D), jnp.float32),
            pltpu.SemaphoreType.DMA,
            pltpu.SemaphoreType.DMA,
            pltpu.SemaphoreType.DMA,
            pltpu.SemaphoreType.DMA,
            pltpu.SemaphoreType.DMA,
            pltpu.SemaphoreType.DMA,
            pltpu.SemaphoreType.DMA,
            pltpu.SemaphoreType.DMA,
        ],
    )
    return f(xp, ai, sflat, cidx3d, cvec)


def _c_body(p_ref, dn_ref, xr_ref, b_ref, lw_ref, lb_ref, h_ref):
    denom = jnp.sum(dn_ref[...], axis=0)[:, None] + 1e-16
    agg = (p_ref[0] + p_ref[1]) / denom + xr_ref[...] + b_ref[...]
    mu = jnp.mean(agg, axis=-1, keepdims=True)
    var = jnp.mean(jnp.square(agg - mu), axis=-1, keepdims=True)
    hn = (agg - mu) * lax.rsqrt(var + 1e-5) * lw_ref[...] + lb_ref[...]
    h_ref[...] = jnp.tanh(hn)


def _run_c(aggr_p, den_p, xr, bias, lnw, lnb):
    return pl.pallas_call(
        _c_body,
        grid=(_NBLK,),
        in_specs=[
            pl.BlockSpec((_NC, _BN, _D), lambda i: (0, i, 0)),
            pl.BlockSpec((_NW, _BN), lambda i: (0, i)),
            pl.BlockSpec((_BN, _D), lambda i: (i, 0)),
            pl.BlockSpec((1, _D), lambda i: (0, 0)),
            pl.BlockSpec((1, _D), lambda i: (0, 0)),
            pl.BlockSpec((1, _D), lambda i: (0, 0)),
        ],
        out_specs=pl.BlockSpec((_BN, _D), lambda i: (i, 0)),
        out_shape=jax.ShapeDtypeStruct((_NP, _D), jnp.float32),
    )(aggr_p, den_p, xr, bias, lnw, lnb)


def _layer(xpad, cidx3d,
           basis, att_r, att, root, bias, ln_w, ln_b):
    att_a = att[0, :_D]
    att_b = att[0, _D:]
    w = jnp.matmul(att_r, basis.reshape(_NB, -1)).reshape(_R, _D, _D)
    wc = w.transpose(1, 0, 2).reshape(_D, _R * _D)
    q = jnp.matmul(w, att_b).T
    ws = jnp.concatenate(
        [root, q, jnp.zeros((_D, _D - _R), jnp.float32)], axis=1)
    aa = att_a.reshape(_D, 1)

    xp = _run_a1(xpad, wc)
    xr, s, a, ma, ms = _run_a2(xpad, ws, aa)

    c = jnp.maximum(0.0, ma[0, 0] + ms[0, 0])
    cvec = jnp.full((16,), c, jnp.float32)

    aggr_p, den_p = _run_sc(
        xp.reshape(_R * _NP, _D), a.reshape(_NP), s.reshape(_NP * _R),
        cidx3d, cvec)

    return _run_c(aggr_p, den_p, xr, bias.reshape(1, _D),
                  ln_w.reshape(1, _D), ln_b.reshape(1, _D))


def kernel(x, edge_index, edge_type,
           basis0, att_r0, att0, root0, bias0, ln_w0, ln_b0,
           basis1, att_r1, att1, root1, bias1, ln_w1, ln_b1):
    xpad = jnp.pad(x, ((0, _NP - _N), (0, 0)))
    src = edge_index[0]
    dst = edge_index[1]
    dstp = jnp.pad(dst, (0, _EP - _E), constant_values=_N)
    srcp = jnp.pad(src, (0, _EP - _E))
    etp = jnp.pad(edge_type, (0, _EP - _E))
    cidx3d = jnp.stack(
        [dstp.reshape(_EP // _CK, _CK),
         (srcp * _R + etp).reshape(_EP // _CK, _CK),
         (etp * _NP + srcp).reshape(_EP // _CK, _CK)], axis=1)

    h1p = _layer(xpad, cidx3d,
                 basis0, att_r0, att0, root0, bias0, ln_w0, ln_b0)
    h2p = _layer(h1p, cidx3d,
                 basis1, att_r1, att1, root1, bias1, ln_w1, ln_b1)

    h1 = h1p[:_N]
    h2 = h2p[:_N]
    return (h2, (h1, h2))


# speedup vs baseline: 1.0270x; 1.0270x over previous
import jax
import jax.numpy as jnp
from jax import lax
from jax.experimental import pallas as pl
from jax.experimental.pallas import tpu as pltpu
from jax.experimental.pallas import tpu_sc as plsc

_N = 10000
_E = 160000
_D = 128
_R = 8
_NB = 4

_NP = 10240
_NBLK = 40
_BN = 256
_NC = 2
_NS = 16
_EPW = 5120
_CK = 80
_ECH = _EPW // _CK
_ECH_A = 104
_ECH_B = 24
_NW = _NC * _NS
_EP = _NW * _EPW
_PSB = _NP // _NS


def _a1_body(x_ref, wc_ref, xp_ref):
    xp_ref[...] = jnp.dot(x_ref[...], wc_ref[...],
                          preferred_element_type=jnp.float32)[None]


def _run_a1(xpad, wc):
    return pl.pallas_call(
        _a1_body,
        grid=(_R,),
        in_specs=[
            pl.BlockSpec((_NP, _D), lambda r: (0, 0)),
            pl.BlockSpec((_D, _D), lambda r: (0, r)),
        ],
        out_specs=pl.BlockSpec((1, _NP, _D), lambda r: (r, 0, 0)),
        out_shape=jax.ShapeDtypeStruct((_R, _NP, _D), jnp.float32),
    )(xpad, wc)


def _a2_body(x_ref, ws_ref, aa_ref, xr_ref, s_ref, a_ref, ma_ref, ms_ref):
    i = pl.program_id(0)
    res = jnp.dot(x_ref[...], ws_ref[...], preferred_element_type=jnp.float32)
    xr = res[:, :_D]
    s = res[:, _D:_D + _R]
    a = jnp.dot(xr, aa_ref[...], preferred_element_type=jnp.float32)
    xr_ref[...] = xr
    s_ref[...] = s
    a_ref[...] = a
    bma = jnp.max(a)
    bms = jnp.max(s)

    @pl.when(i == 0)
    def _():
        ma_ref[0, 0] = bma
        ms_ref[0, 0] = bms

    @pl.when(i > 0)
    def _():
        ma_ref[0, 0] = jnp.maximum(ma_ref[0, 0], bma)
        ms_ref[0, 0] = jnp.maximum(ms_ref[0, 0], bms)


def _run_a2(xpad, ws, aa):
    return pl.pallas_call(
        _a2_body,
        grid=(_NBLK,),
        in_specs=[
            pl.BlockSpec((_BN, _D), lambda i: (i, 0)),
            pl.BlockSpec((_D, 2 * _D), lambda i: (0, 0)),
            pl.BlockSpec((_D, 1), lambda i: (0, 0)),
        ],
        out_specs=[
            pl.BlockSpec((_BN, _D), lambda i: (i, 0)),
            pl.BlockSpec((_BN, _R), lambda i: (i, 0)),
            pl.BlockSpec((_BN, 1), lambda i: (i, 0)),
            pl.BlockSpec(memory_space=pltpu.SMEM),
            pl.BlockSpec(memory_space=pltpu.SMEM),
        ],
        out_shape=[
            jax.ShapeDtypeStruct((_NP, _D), jnp.float32),
            jax.ShapeDtypeStruct((_NP, _R), jnp.float32),
            jax.ShapeDtypeStruct((_NP, 1), jnp.float32),
            jax.ShapeDtypeStruct((1, 1), jnp.float32),
            jax.ShapeDtypeStruct((1, 1), jnp.float32),
        ],
    )(xpad, ws, aa)


def _sc_body(xp_hbm, ai_hbm, s_hbm, cidx_hbm, c_hbm,
             aggr_out, den_out,
             ci0, ci1, a_row0, a_row1, s_row0, s_row1, e_row0, e_row1,
             rows_v0, rows_v1, c_v, den_v, aggr_sp,
             sem_a0, sem_a1, sem_s0, sem_s1, sem_r0, sem_r1,
             sem_w0, sem_w1):
    cid = lax.axis_index("c")
    sid = lax.axis_index("s")
    wid = sid * _NC + cid
    ech = jnp.where(cid == 0, _ECH_A, _ECH_B)
    cbase = jnp.where(cid == 0, 0, _ECH_A * _NS) + sid * ech

    cis = (ci0, ci1)
    a_rows = (a_row0, a_row1)
    s_rows = (s_row0, s_row1)
    e_rows = (e_row0, e_row1)
    rows_vs = (rows_v0, rows_v1)
    sems_a = (sem_a0, sem_a1)
    sems_s = (sem_s0, sem_s1)
    sems_r = (sem_r0, sem_r1)
    sems_w = (sem_w0, sem_w1)

    def zrow(i, c0):
        for f in range(8):
            rows_v0[i, pl.ds(f * 16, 16)] = jnp.zeros((16,), jnp.float32)
        return c0

    lax.fori_loop(0, _CK, zrow, 0)

    def zcopy(i, c0):
        pltpu.sync_copy(rows_v0,
                        aggr_sp.at[pl.ds(sid * _PSB + i * _CK, _CK)])
        return c0

    lax.fori_loop(0, _PSB // _CK, zcopy, 0)

    def zden(i, c0):
        den_v[pl.ds(i * 16, 16)] = jnp.zeros((16,), jnp.float32)
        return c0

    lax.fori_loop(0, _NP // 16, zden, 0)

    pltpu.sync_copy(c_hbm, c_v)
    cvec = c_v[...]

    plsc.subcore_barrier()

    def issue(ch, b):
        pltpu.async_copy(ai_hbm.at[cis[b].at[0]], a_rows[b], sems_a[b])
        pltpu.async_copy(s_hbm.at[cis[b].at[1]], s_rows[b], sems_s[b])
        pltpu.async_copy(xp_hbm.at[cis[b].at[2]], rows_vs[b], sems_r[b])

    pltpu.sync_copy(cidx_hbm.at[cbase], ci0)
    issue(0, 0)

    def halfstep(half, carry):
        for b in (0, 1):
            ch = half * 2 + b
            nb = 1 - b

            @pl.when(ch + 1 < ech)
            def _():
                pltpu.sync_copy(cidx_hbm.at[cbase + ch + 1], cis[nb])

                @pl.when(ch >= 1)
                def _():
                    pltpu.make_async_copy(
                        rows_vs[nb], aggr_sp.at[cis[nb].at[0]],
                        sems_w[nb]).wait()

                issue(ch + 1, nb)

            pltpu.make_async_copy(
                ai_hbm.at[cis[b].at[0]], a_rows[b], sems_a[b]).wait()
            pltpu.make_async_copy(
                s_hbm.at[cis[b].at[1]], s_rows[b], sems_s[b]).wait()
            for k in range(_CK // 16):
                a16 = a_rows[b][pl.ds(k * 16, 16)]
                s16 = s_rows[b][pl.ds(k * 16, 16)]
                d16 = cis[b][0, pl.ds(k * 16, 16)]
                z = a16 + s16
                alpha = jnp.where(z >= 0.0, z, 0.2 * z) - cvec
                e16 = jnp.exp(alpha)
                e_rows[b][pl.ds(k * 16, 16)] = e16
                plsc.addupdate_scatter(den_v, [d16], e16)
            pltpu.make_async_copy(
                xp_hbm.at[cis[b].at[2]], rows_vs[b], sems_r[b]).wait()

            def scale_row(j, c2):
                b16 = plsc.load_gather(
                    e_rows[b], [jnp.full((16,), j, jnp.int32)])
                for f in range(8):
                    rows_vs[b][j, pl.ds(f * 16, 16)] = (
                        rows_vs[b][j, pl.ds(f * 16, 16)] * b16)
                return c2

            lax.fori_loop(0, _CK, scale_row, 0)

            pltpu.async_copy(rows_vs[b], aggr_sp.at[cis[b].at[0]],
                             sems_w[b], add=True)
        return carry

    lax.fori_loop(0, (ech + 1) // 2, halfstep, 0)

    pltpu.make_async_copy(rows_vs[0], aggr_sp.at[cis[0].at[0]],
                          sems_w[0]).wait()
    pltpu.make_async_copy(rows_vs[1], aggr_sp.at[cis[1].at[0]],
                          sems_w[1]).wait()

    plsc.subcore_barrier()

    pltpu.sync_copy(aggr_sp.at[pl.ds(sid * _PSB, _PSB)],
                    aggr_out.at[cid, pl.ds(sid * _PSB, _PSB)])
    pltpu.sync_copy(den_v, den_out.at[wid])


def _run_sc(xp, ai, sflat, cidx3d, cvec):
    mesh = plsc.VectorSubcoreMesh(core_axis_name="c", subcore_axis_name="s")
    f = pl.kernel(
        _sc_body,
        out_type=[
            jax.ShapeDtypeStruct((_NC, _NP, _D), jnp.float32),
            jax.ShapeDtypeStruct((_NW, _NP), jnp.float32),
        ],
        mesh=mesh,
        compiler_params=pltpu.CompilerParams(needs_layout_passes=False),
        scratch_types=[
            pltpu.VMEM((3, _CK), jnp.int32),
            pltpu.VMEM((3, _CK), jnp.int32),
            pltpu.VMEM((_CK,), jnp.float32),
            pltpu.VMEM((_CK,), jnp.float32),
            pltpu.VMEM((_CK,), jnp.float32),
            pltpu.VMEM((_CK,), jnp.float32),
            pltpu.VMEM((_CK,), jnp.float32),
            pltpu.VMEM((_CK,), jnp.float32),
            pltpu.VMEM((_CK, _D), jnp.float32),
            pltpu.VMEM((_CK, _D), jnp.float32),
            pltpu.VMEM((16,), jnp.float32),
            pltpu.VMEM((_NP,), jnp.float32),
            pltpu.VMEM_SHARED((_NP, _D), jnp.float32),
            pltpu.SemaphoreType.DMA,
            pltpu.SemaphoreType.DMA,
            pltpu.SemaphoreType.DMA,
            pltpu.SemaphoreType.DMA,
            pltpu.SemaphoreType.DMA,
            pltpu.SemaphoreType.DMA,
            pltpu.SemaphoreType.DMA,
            pltpu.SemaphoreType.DMA,
        ],
    )
    return f(xp, ai, sflat, cidx3d, cvec)


def _c_body(p_ref, dn_ref, xr_ref, b_ref, lw_ref, lb_ref, h_ref):
    denom = jnp.sum(dn_ref[...], axis=0)[:, None] + 1e-16
    agg = (p_ref[0] + p_ref[1]) / denom + xr_ref[...] + b_ref[...]
    mu = jnp.mean(agg, axis=-1, keepdims=True)
    var = jnp.mean(jnp.square(agg - mu), axis=-1, keepdims=True)
    hn = (agg - mu) * lax.rsqrt(var + 1e-5) * lw_ref[...] + lb_ref[...]
    h_ref[...] = jnp.tanh(hn)


def _run_c(aggr_p, den_p, xr, bias, lnw, lnb):
    return pl.pallas_call(
        _c_body,
        grid=(_NBLK,),
        in_specs=[
            pl.BlockSpec((_NC, _BN, _D), lambda i: (0, i, 0)),
            pl.BlockSpec((_NW, _BN), lambda i: (0, i)),
            pl.BlockSpec((_BN, _D), lambda i: (i, 0)),
            pl.BlockSpec((1, _D), lambda i: (0, 0)),
            pl.BlockSpec((1, _D), lambda i: (0, 0)),
            pl.BlockSpec((1, _D), lambda i: (0, 0)),
        ],
        out_specs=pl.BlockSpec((_BN, _D), lambda i: (i, 0)),
        out_shape=jax.ShapeDtypeStruct((_NP, _D), jnp.float32),
    )(aggr_p, den_p, xr, bias, lnw, lnb)


def _layer(xpad, cidx3d,
           basis, att_r, att, root, bias, ln_w, ln_b):
    att_a = att[0, :_D]
    att_b = att[0, _D:]
    w = jnp.matmul(att_r, basis.reshape(_NB, -1)).reshape(_R, _D, _D)
    wc = w.transpose(1, 0, 2).reshape(_D, _R * _D)
    q = jnp.matmul(w, att_b).T
    ws = jnp.concatenate(
        [root, q, jnp.zeros((_D, _D - _R), jnp.float32)], axis=1)
    aa = att_a.reshape(_D, 1)

    xp = _run_a1(xpad, wc)
    xr, s, a, ma, ms = _run_a2(xpad, ws, aa)

    c = jnp.maximum(0.0, ma[0, 0] + ms[0, 0])
    cvec = jnp.full((16,), c, jnp.float32)

    aggr_p, den_p = _run_sc(
        xp.reshape(_R * _NP, _D), a.reshape(_NP), s.reshape(_NP * _R),
        cidx3d, cvec)

    return _run_c(aggr_p, den_p, xr, bias.reshape(1, _D),
                  ln_w.reshape(1, _D), ln_b.reshape(1, _D))


def kernel(x, edge_index, edge_type,
           basis0, att_r0, att0, root0, bias0, ln_w0, ln_b0,
           basis1, att_r1, att1, root1, bias1, ln_w1, ln_b1):
    xpad = jnp.pad(x, ((0, _NP - _N), (0, 0)))
    src = edge_index[0]
    dst = edge_index[1]
    dstp = jnp.pad(dst, (0, _EP - _E), constant_values=_N)
    srcp = jnp.pad(src, (0, _EP - _E))
    etp = jnp.pad(edge_type, (0, _EP - _E))
    cidx3d = jnp.stack(
        [dstp.reshape(_EP // _CK, _CK),
         (srcp * _R + etp).reshape(_EP // _CK, _CK),
         (etp * _NP + srcp).reshape(_EP // _CK, _CK)], axis=1)

    h1p = _layer(xpad, cidx3d,
                 basis0, att_r0, att0, root0, bias0, ln_w0, ln_b0)
    h2p = _layer(h1p, cidx3d,
                 basis1, att_r1, att1, root1, bias1, ln_w1, ln_b1)

    h1 = h1p[:_N]
    h2 = h2p[:_N]
    return (h2, (h1, h2))
